# Initial kernel scaffold; baseline (speedup 1.0000x reference)
import jax
import jax.numpy as jnp
from jax.experimental import pallas as pl


def kernel(x):
    raise NotImplementedError("write your pallas kernel here")


# speedup vs baseline: 18.6953x; 18.6953x over previous
import jax
import jax.numpy as jnp
from jax.experimental import pallas as pl

_ROWS_PER_BLOCK = 8
_ITERS = 20


def _sparsemax_block(x_ref, o_ref):
    x = x_ref[...]
    m = jnp.max(x, axis=-1, keepdims=True)
    xs = x - m
    lo = jnp.full_like(m, -1.0)
    hi = jnp.zeros_like(m)
    tau = lo

    def body(_, carry):
        lo, hi, tau = carry
        mask = xs > tau
        c = jnp.sum(mask.astype(jnp.float32), axis=-1, keepdims=True)
        s = jnp.sum(jnp.where(mask, xs, 0.0), axis=-1, keepdims=True)
        f = s - c * tau - 1.0
        pos = f > 0.0
        lo = jnp.where(pos, tau, lo)
        hi = jnp.where(pos, hi, tau)
        t_newton = (s - 1.0) / jnp.maximum(c, 1.0)
        ok = (t_newton > lo) & (t_newton < hi)
        tau = jnp.where(ok, t_newton, 0.5 * (lo + hi))
        return lo, hi, tau

    _, _, tau = jax.lax.fori_loop(0, _ITERS, body, (lo, hi, tau))
    o_ref[...] = jnp.maximum(xs - tau, 0.0)


def kernel(x):
    rows, n = x.shape
    r = _ROWS_PER_BLOCK
    return pl.pallas_call(
        _sparsemax_block,
        out_shape=jax.ShapeDtypeStruct(x.shape, x.dtype),
        grid=(rows // r,),
        in_specs=[pl.BlockSpec((r, n), lambda i: (i, 0))],
        out_specs=pl.BlockSpec((r, n), lambda i: (i, 0)),
    )(x)


# speedup vs baseline: 30.0996x; 1.6100x over previous
import jax
import jax.numpy as jnp
from jax.experimental import pallas as pl
from jax.experimental.pallas import tpu as pltpu

_ROWS_PER_BLOCK = 8
_ITERS = 10


def _sparsemax_block(x_ref, o_ref):
    x = x_ref[...]
    m = jnp.max(x, axis=-1, keepdims=True)
    tau = m - 1.0

    def body(_, tau):
        mask = x > tau
        c = jnp.sum(mask.astype(jnp.float32), axis=-1, keepdims=True)
        s = jnp.sum(jnp.where(mask, x, 0.0), axis=-1, keepdims=True)
        return (s - 1.0) / jnp.maximum(c, 1.0)

    tau = jax.lax.fori_loop(0, _ITERS, body, tau)
    o_ref[...] = jnp.maximum(x - tau, 0.0)


def kernel(x):
    rows, n = x.shape
    r = _ROWS_PER_BLOCK
    return pl.pallas_call(
        _sparsemax_block,
        out_shape=jax.ShapeDtypeStruct(x.shape, x.dtype),
        grid=(rows // r,),
        in_specs=[pl.BlockSpec((r, n), lambda i: (i, 0))],
        out_specs=pl.BlockSpec((r, n), lambda i: (i, 0)),
        compiler_params=pltpu.CompilerParams(
            dimension_semantics=("parallel",),
        ),
    )(x)


# speedup vs baseline: 65.8086x; 2.1864x over previous
import jax
import jax.numpy as jnp
from jax.experimental import pallas as pl
from jax.experimental.pallas import tpu as pltpu

_ROWS_PER_BLOCK = 64
_MAX_ITERS = 16


def _sparsemax_block(x_ref, o_ref):
    x = x_ref[...]
    m = jnp.max(x, axis=-1, keepdims=True)
    tau0 = m - 1.0

    def cond(carry):
        it, tau, prev = carry
        return jnp.logical_and(it < _MAX_ITERS, jnp.any(tau != prev))

    def body(carry):
        it, tau, _ = carry
        d = x - tau
        s = jnp.sum(jnp.maximum(d, 0.0), axis=-1, keepdims=True)
        c = jnp.sum((d > 0.0).astype(jnp.float32), axis=-1, keepdims=True)
        new = tau + (s - 1.0) / jnp.maximum(c, 1.0)
        return it + 1, new, tau

    _, tau, _ = jax.lax.while_loop(cond, body, (0, tau0, tau0 - 1.0))
    o_ref[...] = jnp.maximum(x - tau, 0.0)


def kernel(x):
    rows, n = x.shape
    r = _ROWS_PER_BLOCK
    return pl.pallas_call(
        _sparsemax_block,
        out_shape=jax.ShapeDtypeStruct(x.shape, x.dtype),
        grid=(rows // r,),
        in_specs=[pl.BlockSpec((r, n), lambda i: (i, 0))],
        out_specs=pl.BlockSpec((r, n), lambda i: (i, 0)),
        compiler_params=pltpu.CompilerParams(
            dimension_semantics=("parallel",),
        ),
    )(x)
